# baseline (speedup 1.0000x reference)
import functools

import jax
import jax.numpy as jnp
from jax import lax
from jax.experimental import pallas as pl
from jax.experimental.pallas import tpu as pltpu
from jax.experimental.pallas import tpu_sc as plsc

NC = 2
NS = 16
NW = NC * NS
LANES = 16
NBUF = 4
EPS = 1e-12


def _fast_rsqrt(x):
    i = plsc.bitcast(x, jnp.int32)
    i = jnp.int32(0x5F3759DF) - lax.shift_right_logical(i, 1)
    y = plsc.bitcast(i, jnp.float32)
    for _ in range(3):
        y = y * (1.5 - 0.5 * x * y * y)
    return y


def _make_sc_kernel(n_rows, embed, c_rows, n_iter):
    mesh = plsc.VectorSubcoreMesh(
        core_axis_name="c", subcore_axis_name="s", num_cores=NC, num_subcores=NS
    )

    @functools.partial(
        pl.kernel,
        mesh=mesh,
        out_type=jax.ShapeDtypeStruct((n_rows, embed), jnp.float32),
        compiler_params=pltpu.CompilerParams(
            needs_layout_passes=False, use_tc_tiling_on_sc=False
        ),
        scratch_types=[
            [pltpu.VMEM((c_rows,), jnp.int32) for _ in range(NBUF)],
            [pltpu.VMEM((c_rows, embed), jnp.float32) for _ in range(NBUF)],
            pltpu.VMEM((embed,), jnp.float32),
            pltpu.VMEM((embed,), jnp.float32),
            [pltpu.SemaphoreType.DMA for _ in range(NBUF)],
            [pltpu.SemaphoreType.DMA for _ in range(NBUF)],
            [pltpu.SemaphoreType.DMA for _ in range(NBUF)],
        ],
    )
    def body(ids_hbm, table_hbm, gamma_hbm, beta_hbm, out_hbm,
             idxbs, bufs, gv, bv, sxs, sis, sos):
        w = lax.axis_index("s") * NC + lax.axis_index("c")
        pltpu.sync_copy(gamma_hbm, gv)
        pltpu.sync_copy(beta_hbm, bv)

        lane = lax.iota(jnp.int32, LANES)
        inv_e = jnp.float32(1.0 / embed)
        n_groups = c_rows // LANES

        def compute(buf):
            gvecs = [gv[pl.ds(k * LANES, LANES)] for k in range(embed // LANES)]
            bvecs = [bv[pl.ds(k * LANES, LANES)] for k in range(embed // LANES)]

            @plsc.parallel_loop(0, n_groups, unroll=2)
            def ln_group(g):
                ridx = g * LANES + lane
                s0 = jnp.zeros((LANES,), jnp.float32)
                s1 = jnp.zeros((LANES,), jnp.float32)
                ss0 = jnp.zeros((LANES,), jnp.float32)
                ss1 = jnp.zeros((LANES,), jnp.float32)
                for c in range(0, embed, 2):
                    cv0 = jnp.full((LANES,), c, jnp.int32)
                    cv1 = jnp.full((LANES,), c + 1, jnp.int32)
                    x0 = plsc.load_gather(buf, [ridx, cv0])
                    x1 = plsc.load_gather(buf, [ridx, cv1])
                    s0 = s0 + x0
                    s1 = s1 + x1
                    ss0 = ss0 + x0 * x0
                    ss1 = ss1 + x1 * x1
                mean = (s0 + s1) * inv_e
                var = jnp.maximum((ss0 + ss1) * inv_e - mean * mean, 0.0)
                rstd = _fast_rsqrt(var + EPS)
                nmean = mean * rstd
                for c in range(embed):
                    cv = jnp.full((LANES,), c, jnp.int32)
                    x = plsc.load_gather(buf, [ridx, cv])
                    gc = gvecs[c // LANES][c % LANES]
                    bc = bvecs[c // LANES][c % LANES]
                    y = (x * rstd - nmean) * gc + bc
                    plsc.store_scatter(buf, [ridx, cv], y)

        def start_idx(i, b):
            pltpu.async_copy(ids_hbm.at[w, i], idxbs[b], sxs[b])

        def wait_idx(i, b):
            pltpu.make_async_copy(ids_hbm.at[w, i], idxbs[b], sxs[b]).wait()

        def start_in(b):
            pltpu.async_copy(table_hbm.at[idxbs[b]], bufs[b], sis[b])

        def wait_in(b):
            pltpu.make_async_copy(table_hbm.at[idxbs[b]], bufs[b], sis[b]).wait()

        def out_slice(i):
            return out_hbm.at[pl.ds((w * n_iter + i) * c_rows, c_rows)]

        start_idx(0, 0)
        start_idx(1, 1)
        start_idx(2, 2)
        wait_idx(0, 0)
        start_in(0)
        wait_idx(1, 1)
        start_in(1)

        def step(i, b):
            wait_in(b)

            b3 = (b + 3) % NBUF

            @pl.when(i + 3 < n_iter)
            def _():
                start_idx(i + 3, b3)

            compute(bufs[b])
            pltpu.async_copy(bufs[b], out_slice(i), sos[b])

            j = i + 2
            b2 = (b + 2) % NBUF

            @pl.when(j < n_iter)
            def _():
                @pl.when(j >= NBUF)
                def _():
                    pltpu.make_async_copy(
                        bufs[b2], out_slice(j - NBUF), sos[b2]
                    ).wait()

                wait_idx(j, b2)
                start_in(b2)

        def outer(o, _):
            for b in range(NBUF):
                step(o * NBUF + b, b)
            return 0

        lax.fori_loop(0, n_iter // NBUF, outer, 0)
        for b in range(NBUF):
            i = n_iter - NBUF + b
            pltpu.make_async_copy(bufs[b], out_slice(i), sos[b]).wait()

    return body


def kernel(input_ids, table, gamma, beta):
    b, s = input_ids.shape
    vocab, embed = table.shape
    n = b * s
    c_rows = 256
    n_iter = n // (NW * c_rows)
    assert n == NW * n_iter * c_rows and n_iter % NBUF == 0

    ids = input_ids.reshape(NW, n_iter, c_rows).astype(jnp.int32)
    sc = _make_sc_kernel(n, embed, c_rows, n_iter)
    out = sc(ids, table, gamma, beta)
    return out.reshape(b, s, embed)


# speedup vs baseline: 2.5989x; 2.5989x over previous
import functools

import jax
import jax.numpy as jnp
from jax import lax
from jax.experimental import pallas as pl
from jax.experimental.pallas import tpu as pltpu
from jax.experimental.pallas import tpu_sc as plsc

NC = 2
NS = 16
NW = NC * NS
LANES = 16
NBUF = 4
EPS = 1e-12


def _fast_rsqrt(x):
    i = plsc.bitcast(x, jnp.int32)
    i = jnp.int32(0x5F3759DF) - lax.shift_right_logical(i, 1)
    y = plsc.bitcast(i, jnp.float32)
    for _ in range(2):
        y = y * (1.5 - 0.5 * x * y * y)
    return y


def _make_sc_kernel(n_rows, embed, c_rows, n_iter):
    mesh = plsc.VectorSubcoreMesh(
        core_axis_name="c", subcore_axis_name="s", num_cores=NC, num_subcores=NS
    )

    @functools.partial(
        pl.kernel,
        mesh=mesh,
        out_type=jax.ShapeDtypeStruct((n_rows, embed), jnp.float32),
        compiler_params=pltpu.CompilerParams(
            needs_layout_passes=False, use_tc_tiling_on_sc=False
        ),
        scratch_types=[
            [pltpu.VMEM((c_rows,), jnp.int32) for _ in range(NBUF)],
            [pltpu.VMEM((c_rows, embed), jnp.float32) for _ in range(NBUF)],
            pltpu.VMEM((embed,), jnp.float32),
            pltpu.VMEM((embed,), jnp.float32),
            [pltpu.SemaphoreType.DMA for _ in range(NBUF)],
            [pltpu.SemaphoreType.DMA for _ in range(NBUF)],
            [pltpu.SemaphoreType.DMA for _ in range(NBUF)],
        ],
    )
    def body(ids_hbm, table_hbm, gamma_hbm, beta_hbm, out_hbm,
             idxbs, bufs, gv, bv, sxs, sis, sos):
        w = lax.axis_index("s") * NC + lax.axis_index("c")
        pltpu.sync_copy(gamma_hbm, gv)
        pltpu.sync_copy(beta_hbm, bv)

        lane = lax.iota(jnp.int32, LANES)
        inv_e = jnp.float32(1.0 / embed)
        n_groups = c_rows // LANES

        perms = [lane ^ (1 << t) for t in range(4)]

        def xlsum(v):
            for p in perms:
                v = v + jnp.take_along_axis(
                    v, p, axis=0, mode="promise_in_bounds"
                )
            return v

        nq = embed // LANES

        def compute(buf):
            gvecs = [gv[pl.ds(k * LANES, LANES)] for k in range(nq)]
            bvecs = [bv[pl.ds(k * LANES, LANES)] for k in range(nq)]

            @plsc.parallel_loop(0, c_rows, unroll=4)
            def ln_row(r):
                xs = [buf[r, pl.ds(k * LANES, LANES)] for k in range(nq)]
                s = xs[0] + xs[1] + xs[2] + xs[3]
                sq = (xs[0] * xs[0] + xs[1] * xs[1]
                      + xs[2] * xs[2] + xs[3] * xs[3])
                mean = xlsum(s) * inv_e
                var = jnp.maximum(xlsum(sq) * inv_e - mean * mean, 0.0)
                rstd = _fast_rsqrt(var + EPS)
                m2 = mean * rstd
                for k in range(nq):
                    y = (xs[k] * rstd - m2) * gvecs[k] + bvecs[k]
                    buf[r, pl.ds(k * LANES, LANES)] = y

        def start_idx(i, b):
            pltpu.async_copy(ids_hbm.at[w, i], idxbs[b], sxs[b])

        def wait_idx(i, b):
            pltpu.make_async_copy(ids_hbm.at[w, i], idxbs[b], sxs[b]).wait()

        def start_in(b):
            pltpu.async_copy(table_hbm.at[idxbs[b]], bufs[b], sis[b])

        def wait_in(b):
            pltpu.make_async_copy(table_hbm.at[idxbs[b]], bufs[b], sis[b]).wait()

        def out_slice(i):
            return out_hbm.at[pl.ds((w * n_iter + i) * c_rows, c_rows)]

        start_idx(0, 0)
        start_idx(1, 1)
        start_idx(2, 2)
        wait_idx(0, 0)
        start_in(0)
        wait_idx(1, 1)
        start_in(1)

        def step(i, b):
            wait_in(b)

            b3 = (b + 3) % NBUF

            @pl.when(i + 3 < n_iter)
            def _():
                start_idx(i + 3, b3)

            compute(bufs[b])
            pltpu.async_copy(bufs[b], out_slice(i), sos[b])

            j = i + 2
            b2 = (b + 2) % NBUF

            @pl.when(j < n_iter)
            def _():
                @pl.when(j >= NBUF)
                def _():
                    pltpu.make_async_copy(
                        bufs[b2], out_slice(j - NBUF), sos[b2]
                    ).wait()

                wait_idx(j, b2)
                start_in(b2)

        def outer(o, _):
            for b in range(NBUF):
                step(o * NBUF + b, b)
            return 0

        lax.fori_loop(0, n_iter // NBUF, outer, 0)
        for b in range(NBUF):
            i = n_iter - NBUF + b
            pltpu.make_async_copy(bufs[b], out_slice(i), sos[b]).wait()

    return body


def kernel(input_ids, table, gamma, beta):
    b, s = input_ids.shape
    vocab, embed = table.shape
    n = b * s
    c_rows = 256
    n_iter = n // (NW * c_rows)
    assert n == NW * n_iter * c_rows and n_iter % NBUF == 0

    ids = input_ids.reshape(NW, n_iter, c_rows).astype(jnp.int32)
    sc = _make_sc_kernel(n, embed, c_rows, n_iter)
    out = sc(ids, table, gamma, beta)
    return out.reshape(b, s, embed)
